# baseline (speedup 1.0000x reference)
import functools

import jax
import jax.numpy as jnp
from jax import lax
from jax.experimental import pallas as pl
from jax.experimental.pallas import tpu as pltpu
from jax.experimental.pallas import tpu_sc as plsc

_NC = 2
_NS = 16
_CH = 128
_ROWS = 632
_PADROWS = 112


def _make_mega_kernel(N, N1, D, NCH):
    IB = 40
    assert NCH % IB == 0
    NB = NCH // IB
    assert _ROWS * _NS == N1
    NZC = (_ROWS + _CH - 1) // _CH

    @functools.partial(
        pl.kernel,
        out_type=[
            jax.ShapeDtypeStruct((N, D), jnp.float32),
            jax.ShapeDtypeStruct((N, D), jnp.float32),
            jax.ShapeDtypeStruct((N1, D), jnp.float32),
            jax.ShapeDtypeStruct((N1, D), jnp.float32),
        ],
        mesh=_mesh(),
        scratch_types=[
            pltpu.VMEM((IB, _CH), jnp.int32),
            pltpu.VMEM((IB, _CH), jnp.int32),
            pltpu.VMEM((2, _CH, D), jnp.float32),
            pltpu.VMEM((_CH,), jnp.float32),
            pltpu.VMEM((640,), jnp.float32),
            pltpu.VMEM((5, _CH), jnp.int32),
            pltpu.VMEM_SHARED((N1, D), jnp.float32),
            pltpu.VMEM_SHARED((N1,), jnp.float32),
            pltpu.VMEM_SHARED((N1,), jnp.float32),
            pltpu.SemaphoreType.DMA,
            pltpu.SemaphoreType.DMA,
        ],
    )
    def mega_kernel(y_ui, y_iu, src_ui, dst_ui, src_iu, dst_iu, zvec, zblk,
                    lut,
                    out_item, out_user, tab_ui, tab_iu,
                    sidx, didx, rows, ones, scl, scli, acc, hs, ht,
                    gsem, ssem):
        c = lax.axis_index("c")
        s = lax.axis_index("s")

        for j in range(_CH // 16):
            ones[pl.ds(j * 16, 16)] = jnp.ones((16,), jnp.float32)

        @pl.when(s == 0)
        def _():
            pltpu.sync_copy(zvec, hs)

        @pl.when(s == 1)
        def _():
            pltpu.sync_copy(zvec, ht)

        pltpu.sync_copy(zblk, rows.at[0])
        r0 = pl.multiple_of(s * _ROWS, 8)
        for t in range(NZC):
            off = pl.multiple_of(
                r0 + jnp.minimum(t * _CH, _ROWS - _CH), 8)
            pltpu.sync_copy(rows.at[0], acc.at[pl.ds(off, _CH)])
        plsc.subcore_barrier()

        def hist(src_hbm, dst_hbm):
            def outer(o, _):
                ob = pl.multiple_of(o * IB, 8)
                pltpu.sync_copy(src_hbm.at[s, pl.ds(ob, IB)], sidx)
                pltpu.sync_copy(dst_hbm.at[s, pl.ds(ob, IB)], didx)
                pltpu.async_copy(ones, hs.at[sidx.at[0]], ssem, add=True)
                pltpu.async_copy(ones, ht.at[didx.at[0]], ssem, add=True)

                def body(j, _):
                    pltpu.async_copy(ones, hs.at[sidx.at[j]], ssem, add=True)
                    pltpu.async_copy(ones, ht.at[didx.at[j]], ssem, add=True)
                    pltpu.make_async_copy(ones, hs.at[sidx.at[j - 1]],
                                          ssem).wait()
                    pltpu.make_async_copy(ones, ht.at[didx.at[j - 1]],
                                          ssem).wait()
                    return 0

                lax.fori_loop(1, IB, body, 0)
                pltpu.make_async_copy(ones, hs.at[sidx.at[IB - 1]],
                                      ssem).wait()
                pltpu.make_async_copy(ones, ht.at[didx.at[IB - 1]],
                                      ssem).wait()
                return 0

            lax.fori_loop(0, NB, outer, 0)

        def fill_scale(h_ref, base):
            pltpu.sync_copy(h_ref.at[pl.ds(base, _ROWS)],
                            scl.at[pl.ds(0, _ROWS)])
            lane = lax.iota(jnp.int32, 16)
            for v in range(640 // 16):
                dg = scl[pl.ds(v * 16, 16)]
                if (v + 1) * 16 > _ROWS:
                    dg = jnp.where(lane < _ROWS - v * 16, dg, 0.0)
                scli[v // 8, pl.ds((v % 8) * 16, 16)] = dg.astype(jnp.int32)
            for q in range(5):
                pltpu.async_copy(lut.at[scli.at[q]],
                                 scl.at[pl.ds(q * _CH, _CH)], gsem).wait()

        def scale_rows(bi, lbase, nr, do_relu):
            def srow(r, _):
                g = scl[pl.ds(lbase + (r // 16) * 16, 16)]
                sv = lax.gather(
                    g, jnp.full((16, 1), 0, jnp.int32) + (r % 16),
                    dimension_numbers=lax.GatherDimensionNumbers(
                        offset_dims=(), collapsed_slice_dims=(0,),
                        start_index_map=(0,)),
                    slice_sizes=(1,),
                    mode=lax.GatherScatterMode.PROMISE_IN_BOUNDS)
                for k in range(D // 16):
                    x = rows[bi, r, pl.ds(k * 16, 16)] * sv
                    if do_relu:
                        x = jnp.maximum(x, 0.0)
                    rows[bi, r, pl.ds(k * 16, 16)] = x
                return 0

            lax.fori_loop(0, nr, srow, 0)

        def scale_table(y_hbm, tab_hbm):
            fill_scale(hs, r0)
            for t in range(NZC):
                nr = min(_CH, _ROWS - t * _CH)
                goff = pl.multiple_of(r0 + t * _CH, 8)
                pltpu.sync_copy(y_hbm.at[pl.ds(goff, nr)],
                                rows.at[0, pl.ds(0, nr)])
                scale_rows(0, t * _CH, nr, False)
                pltpu.sync_copy(rows.at[0, pl.ds(0, nr)],
                                tab_hbm.at[pl.ds(goff, nr)])

        def gsa(tab, src_hbm, dst_hbm):
            H = _CH // 2

            def issue_g(j, b):
                pltpu.async_copy(tab.at[sidx.at[j, pl.ds(0, H)]],
                                 rows.at[b, pl.ds(0, H)], gsem)
                pltpu.async_copy(tab.at[sidx.at[j, pl.ds(H, H)]],
                                 rows.at[b, pl.ds(H, H)], gsem)

            def wait_g(b):
                for _ in range(2):
                    pltpu.make_async_copy(tab.at[sidx.at[0, pl.ds(0, H)]],
                                          rows.at[b, pl.ds(0, H)],
                                          gsem).wait()

            def issue_s(j, b):
                pltpu.async_copy(rows.at[b], acc.at[didx.at[j]], ssem,
                                 add=True)

            def wait_s(b):
                pltpu.make_async_copy(rows.at[b], acc.at[didx.at[0]],
                                      ssem).wait()

            def outer(o, _):
                ob = pl.multiple_of(o * IB, 8)
                pltpu.sync_copy(src_hbm.at[s, pl.ds(ob, IB)], sidx)
                pltpu.sync_copy(dst_hbm.at[s, pl.ds(ob, IB)], didx)

                issue_g(0, 0)
                wait_g(0)
                issue_s(0, 0)
                issue_g(1, 1)

                def body(h, _):
                    for bi in range(2):
                        j = 1 + h * 2 + bi
                        b = (1 + bi) % 2
                        wait_g(b)
                        issue_s(j, b)
                        wait_s(1 - b)
                        issue_g(j + 1, 1 - b)
                    return 0

                lax.fori_loop(0, (IB - 2) // 2, body, 0)
                wait_g(1)
                issue_s(IB - 1, 1)
                wait_s(0)
                wait_s(1)
                return 0

            lax.fori_loop(0, NB, outer, 0)

        def emit(out_hbm):
            o0 = pl.multiple_of(jnp.minimum(s * _ROWS, N - _ROWS), 8)
            fill_scale(ht, o0)
            for t in range(NZC):
                loff = min(t * _CH, _ROWS - _CH)
                goff = pl.multiple_of(o0 + loff, 8)
                pltpu.sync_copy(acc.at[pl.ds(goff, _CH)], rows.at[0])
                scale_rows(0, loff, _CH, True)
                pltpu.sync_copy(rows.at[0], out_hbm.at[pl.ds(goff, _CH)])

        def relation(y_hbm, tab_hbm, src_hbm, dst_hbm, out_hbm):
            hist(src_hbm, dst_hbm)
            plsc.subcore_barrier()
            scale_table(y_hbm, tab_hbm)
            plsc.subcore_barrier()
            gsa(tab_hbm, src_hbm, dst_hbm)
            plsc.subcore_barrier()
            emit(out_hbm)

        @pl.when(c == 0)
        def _():
            relation(y_ui, tab_ui, src_ui, dst_ui, out_item)

        @pl.when(c == 1)
        def _():
            relation(y_iu, tab_iu, src_iu, dst_iu, out_user)

    return mega_kernel


def _mesh():
    return plsc.VectorSubcoreMesh(core_axis_name="c", subcore_axis_name="s")


def _mm(x, W, out_rows):
    N, D_in = x.shape
    D_out = W.shape[1]
    B = 2000
    assert N % B == 0

    def body(x_ref, w_ref, o_ref):
        o_ref[...] = jnp.dot(x_ref[...], w_ref[...],
                             preferred_element_type=jnp.float32)

    return pl.pallas_call(
        body,
        grid=(N // B,),
        in_specs=[
            pl.BlockSpec((B, D_in), lambda i: (i, 0)),
            pl.BlockSpec((D_in, D_out), lambda i: (0, 0)),
        ],
        out_specs=pl.BlockSpec((B, D_out), lambda i: (i, 0)),
        out_shape=jax.ShapeDtypeStruct((out_rows, D_out), jnp.float32),
    )(x, W)


def kernel(x_user, x_item, edge_index_user_item, edge_index_item_user,
           W_ui_src, W_ui_tgt, W_iu_src, W_iu_tgt):
    n_user, D = x_user.shape
    n_item = x_item.shape[0]
    assert n_user == n_item
    N = n_user
    N1 = N + _PADROWS
    E = edge_index_user_item.shape[1]
    grain = _NS * _CH * 40
    Epad = ((E + grain - 1) // grain) * grain
    NCH = Epad // (_NS * _CH)

    pad = Epad - E
    sent = (jnp.arange(pad, dtype=jnp.int32) % _PADROWS) + N

    def prep(e):
        e = e.astype(jnp.int32)
        src = jnp.concatenate([e[0], sent]).reshape(_NS, NCH, _CH)
        dst = jnp.concatenate([e[1], sent]).reshape(_NS, NCH, _CH)
        return src, dst

    src_ui, dst_ui = prep(edge_index_user_item)
    src_iu, dst_iu = prep(edge_index_item_user)

    zvec = jnp.zeros((N1,), jnp.float32)
    zblk = jnp.zeros((_CH, D), jnp.float32)

    dr = jnp.arange(E + 8, dtype=jnp.float32)
    lut = jnp.where(dr > 0.0, lax.rsqrt(jnp.maximum(dr, 1.0)), 0.0)

    y_ui = _mm(x_user, W_ui_src, N1)
    y_iu = _mm(x_item, W_iu_src, N1)

    mega = _make_mega_kernel(N, N1, D, NCH)
    out_item, out_user, _, _ = mega(y_ui, y_iu,
                                    src_ui, dst_ui, src_iu, dst_iu,
                                    zvec, zblk, lut)
    return (out_user, out_item)


# speedup vs baseline: 1.5149x; 1.5149x over previous
import functools

import jax
import jax.numpy as jnp
from jax import lax
from jax.experimental import pallas as pl
from jax.experimental.pallas import tpu as pltpu
from jax.experimental.pallas import tpu_sc as plsc

_NC = 2
_NS = 16
_CH = 128
_PADROWS = 512
_ROWS = 632


def _mesh():
    return plsc.VectorSubcoreMesh(core_axis_name="c", subcore_axis_name="s")


def _make_deg_kernel(N, N1, NCH):
    @functools.partial(
        pl.kernel,
        out_type=[jax.ShapeDtypeStruct((N1,), jnp.float32) for _ in range(4)],
        mesh=_mesh(),
        scratch_types=[
            pltpu.VMEM((NCH, _CH), jnp.int32),
            pltpu.VMEM((NCH, _CH), jnp.int32),
            pltpu.VMEM((_CH,), jnp.float32),
            pltpu.VMEM_SHARED((N1,), jnp.float32),
            pltpu.VMEM_SHARED((N1,), jnp.float32),
            pltpu.SemaphoreType.DMA,
        ],
    )
    def deg_kernel(src_ui, dst_ui, src_iu, dst_iu, zvec,
                   d_su, d_di, d_si, d_du,
                   sidx, didx, ones, hs, ht, ssem):
        c = lax.axis_index("c")
        s = lax.axis_index("s")

        for j in range(_CH // 16):
            ones[pl.ds(j * 16, 16)] = jnp.ones((16,), jnp.float32)

        @pl.when(s == 0)
        def _():
            pltpu.sync_copy(zvec, hs)

        @pl.when(s == 1)
        def _():
            pltpu.sync_copy(zvec, ht)

        plsc.subcore_barrier()

        def accumulate(src_hbm, dst_hbm):
            pltpu.sync_copy(src_hbm.at[s], sidx)
            pltpu.sync_copy(dst_hbm.at[s], didx)

            pltpu.async_copy(ones, hs.at[sidx.at[0]], ssem, add=True)
            pltpu.async_copy(ones, ht.at[didx.at[0]], ssem, add=True)

            def body(j, _):
                pltpu.async_copy(ones, hs.at[sidx.at[j]], ssem, add=True)
                pltpu.async_copy(ones, ht.at[didx.at[j]], ssem, add=True)
                pltpu.make_async_copy(ones, hs.at[sidx.at[j - 1]], ssem).wait()
                pltpu.make_async_copy(ones, ht.at[didx.at[j - 1]], ssem).wait()
                return 0

            lax.fori_loop(1, NCH, body, 0)
            last = NCH - 1
            pltpu.make_async_copy(ones, hs.at[sidx.at[last]], ssem).wait()
            pltpu.make_async_copy(ones, ht.at[didx.at[last]], ssem).wait()

        @pl.when(c == 0)
        def _():
            accumulate(src_ui, dst_ui)

        @pl.when(c == 1)
        def _():
            accumulate(src_iu, dst_iu)

        plsc.subcore_barrier()

        @pl.when((c == 0) & (s == 0))
        def _():
            pltpu.sync_copy(hs, d_su)

        @pl.when((c == 0) & (s == 1))
        def _():
            pltpu.sync_copy(ht, d_di)

        @pl.when((c == 1) & (s == 0))
        def _():
            pltpu.sync_copy(hs, d_si)

        @pl.when((c == 1) & (s == 1))
        def _():
            pltpu.sync_copy(ht, d_du)

    return deg_kernel


def _make_gsa_kernel(N, N1, D, NCH):
    IB = 40
    assert NCH % IB == 0
    NB = NCH // IB

    @functools.partial(
        pl.kernel,
        out_type=[
            jax.ShapeDtypeStruct((N, D), jnp.float32),
            jax.ShapeDtypeStruct((N, D), jnp.float32),
        ],
        mesh=_mesh(),
        scratch_types=[
            pltpu.VMEM((IB, _CH), jnp.int32),
            pltpu.VMEM((IB, _CH), jnp.int32),
            pltpu.VMEM((2, _CH, D), jnp.float32),
            pltpu.VMEM_SHARED((N1, D), jnp.float32),
            pltpu.SemaphoreType.DMA,
            pltpu.SemaphoreType.DMA,
        ],
    )
    def gsa_kernel(tab_ui, tab_iu, src_ui, dst_ui, src_iu, dst_iu, zblk,
                   acc_item, acc_user,
                   sidx, didx, rows, acc, gsem, ssem):
        c = lax.axis_index("c")
        s = lax.axis_index("s")

        pltpu.sync_copy(zblk, rows.at[0])
        rz = jnp.minimum(s * _ROWS, N1 - _ROWS)
        nzc = (_ROWS + _CH - 1) // _CH
        for t in range(nzc):
            off = pl.multiple_of(
                rz + jnp.minimum(t * _CH, _ROWS - _CH), 8)
            pltpu.sync_copy(rows.at[0], acc.at[pl.ds(off, _CH)])
        plsc.subcore_barrier()

        def relation(tab, src_hbm, dst_hbm):
            def outer(o, _):
                ob = pl.multiple_of(o * IB, 8)
                pltpu.sync_copy(src_hbm.at[s, pl.ds(ob, IB)], sidx)
                pltpu.sync_copy(dst_hbm.at[s, pl.ds(ob, IB)], didx)
                H = _CH // 2

                def issue_g(j, b):
                    pltpu.async_copy(tab.at[sidx.at[j, pl.ds(0, H)]],
                                     rows.at[b, pl.ds(0, H)], gsem)
                    pltpu.async_copy(tab.at[sidx.at[j, pl.ds(H, H)]],
                                     rows.at[b, pl.ds(H, H)], gsem)

                def wait_g(b):
                    for _ in range(2):
                        pltpu.make_async_copy(tab.at[sidx.at[0, pl.ds(0, H)]],
                                              rows.at[b, pl.ds(0, H)],
                                              gsem).wait()

                def issue_s(j, b):
                    pltpu.async_copy(rows.at[b], acc.at[didx.at[j]], ssem,
                                     add=True)

                def wait_s(b):
                    pltpu.make_async_copy(rows.at[b], acc.at[didx.at[0]],
                                          ssem).wait()

                issue_g(0, 0)
                wait_g(0)
                issue_s(0, 0)
                issue_g(1, 1)

                def body(h, _):
                    for bi in range(2):
                        j = 1 + h * 2 + bi
                        b = (1 + bi) % 2
                        wait_g(b)
                        issue_s(j, b)
                        wait_s(1 - b)
                        issue_g(j + 1, 1 - b)
                    return 0

                lax.fori_loop(0, (IB - 2) // 2, body, 0)
                wait_g(1)
                issue_s(IB - 1, 1)
                wait_s(0)
                wait_s(1)
                return 0

            lax.fori_loop(0, NB, outer, 0)

        @pl.when(c == 0)
        def _():
            relation(tab_ui, src_ui, dst_ui)

        @pl.when(c == 1)
        def _():
            relation(tab_iu, src_iu, dst_iu)

        plsc.subcore_barrier()

        ro = pl.multiple_of(jnp.minimum(s * _ROWS, N - _ROWS), 8)

        @pl.when(c == 0)
        def _():
            pltpu.sync_copy(acc.at[pl.ds(ro, _ROWS)],
                            acc_item.at[pl.ds(ro, _ROWS)])

        @pl.when(c == 1)
        def _():
            pltpu.sync_copy(acc.at[pl.ds(ro, _ROWS)],
                            acc_user.at[pl.ds(ro, _ROWS)])

    return gsa_kernel


def _mm_scale(x, W, deg, out_rows):
    N, D_in = x.shape
    D_out = W.shape[1]
    B = 2000
    assert N % B == 0

    def body(x_ref, w_ref, deg_ref, o_ref):
        dg = deg_ref[...]
        scale = jnp.where(dg > 0.0, lax.rsqrt(dg), 0.0)
        o_ref[...] = jnp.dot(x_ref[...], w_ref[...],
                             preferred_element_type=jnp.float32) * scale

    return pl.pallas_call(
        body,
        grid=(N // B,),
        in_specs=[
            pl.BlockSpec((B, D_in), lambda i: (i, 0)),
            pl.BlockSpec((D_in, D_out), lambda i: (0, 0)),
            pl.BlockSpec((B, 1), lambda i: (i, 0)),
        ],
        out_specs=pl.BlockSpec((B, D_out), lambda i: (i, 0)),
        out_shape=jax.ShapeDtypeStruct((out_rows, D_out), jnp.float32),
    )(x, W, deg[:, None])


def _scale_relu(acc, deg):
    N, D = acc.shape
    B = 2000
    assert N % B == 0

    def body(a_ref, deg_ref, o_ref):
        dg = deg_ref[...]
        scale = jnp.where(dg > 0.0, lax.rsqrt(dg), 0.0)
        o_ref[...] = jnp.maximum(a_ref[...] * scale, 0.0)

    return pl.pallas_call(
        body,
        grid=(N // B,),
        in_specs=[
            pl.BlockSpec((B, D), lambda i: (i, 0)),
            pl.BlockSpec((B, 1), lambda i: (i, 0)),
        ],
        out_specs=pl.BlockSpec((B, D), lambda i: (i, 0)),
        out_shape=jax.ShapeDtypeStruct((N, D), jnp.float32),
    )(acc, deg[:, None])


def kernel(x_user, x_item, edge_index_user_item, edge_index_item_user,
           W_ui_src, W_ui_tgt, W_iu_src, W_iu_tgt):
    n_user, D = x_user.shape
    n_item = x_item.shape[0]
    assert n_user == n_item
    N = n_user
    N1 = N + _PADROWS
    E = edge_index_user_item.shape[1]
    grain = _NS * _CH * 40
    Epad = ((E + grain - 1) // grain) * grain
    NCH = Epad // (_NS * _CH)

    pad = Epad - E
    sent = (jnp.arange(pad, dtype=jnp.int32) % _PADROWS) + N

    def prep(e):
        e = e.astype(jnp.int32)
        src = jnp.concatenate([e[0], sent]).reshape(_NS, NCH, _CH)
        dst = jnp.concatenate([e[1], sent]).reshape(_NS, NCH, _CH)
        return src, dst

    src_ui, dst_ui = prep(edge_index_user_item)
    src_iu, dst_iu = prep(edge_index_item_user)

    zvec = jnp.zeros((N1,), jnp.float32)
    zblk = jnp.zeros((_CH, D), jnp.float32)

    deg_kernel = _make_deg_kernel(N, N1, NCH)
    d_su, d_di, d_si, d_du = deg_kernel(src_ui, dst_ui, src_iu, dst_iu, zvec)
    d_su, d_di, d_si, d_du = (d[:N] for d in (d_su, d_di, d_si, d_du))

    tab_ui = _mm_scale(x_user, W_ui_src, d_su, N1)
    tab_iu = _mm_scale(x_item, W_iu_src, d_si, N1)

    gsa_kernel = _make_gsa_kernel(N, N1, D, NCH)
    acc_item, acc_user = gsa_kernel(tab_ui, tab_iu,
                                    src_ui, dst_ui, src_iu, dst_iu, zblk)

    out_item = _scale_relu(acc_item, d_di)
    out_user = _scale_relu(acc_user, d_du)
    return (out_user, out_item)


# speedup vs baseline: 1.5221x; 1.0047x over previous
import functools

import jax
import jax.numpy as jnp
from jax import lax
from jax.experimental import pallas as pl
from jax.experimental.pallas import tpu as pltpu
from jax.experimental.pallas import tpu_sc as plsc

_NC = 2
_NS = 16
_CH = 128
_PADROWS = 64
_ROWS = 632


def _mesh():
    return plsc.VectorSubcoreMesh(core_axis_name="c", subcore_axis_name="s")


def _make_deg_kernel(N, N1, NCH):
    @functools.partial(
        pl.kernel,
        out_type=[jax.ShapeDtypeStruct((N1,), jnp.float32) for _ in range(4)],
        mesh=_mesh(),
        scratch_types=[
            pltpu.VMEM((NCH, _CH), jnp.int32),
            pltpu.VMEM((NCH, _CH), jnp.int32),
            pltpu.VMEM((_CH,), jnp.float32),
            pltpu.VMEM_SHARED((N1,), jnp.float32),
            pltpu.VMEM_SHARED((N1,), jnp.float32),
            pltpu.SemaphoreType.DMA,
        ],
    )
    def deg_kernel(src_ui, dst_ui, src_iu, dst_iu, zvec,
                   d_su, d_di, d_si, d_du,
                   sidx, didx, ones, hs, ht, ssem):
        c = lax.axis_index("c")
        s = lax.axis_index("s")

        for j in range(_CH // 16):
            ones[pl.ds(j * 16, 16)] = jnp.ones((16,), jnp.float32)

        @pl.when(s == 0)
        def _():
            pltpu.sync_copy(zvec, hs)

        @pl.when(s == 1)
        def _():
            pltpu.sync_copy(zvec, ht)

        plsc.subcore_barrier()

        def accumulate(src_hbm, dst_hbm):
            pltpu.sync_copy(src_hbm.at[s], sidx)
            pltpu.sync_copy(dst_hbm.at[s], didx)

            pltpu.async_copy(ones, hs.at[sidx.at[0]], ssem, add=True)
            pltpu.async_copy(ones, ht.at[didx.at[0]], ssem, add=True)

            def body(j, _):
                pltpu.async_copy(ones, hs.at[sidx.at[j]], ssem, add=True)
                pltpu.async_copy(ones, ht.at[didx.at[j]], ssem, add=True)
                pltpu.make_async_copy(ones, hs.at[sidx.at[j - 1]], ssem).wait()
                pltpu.make_async_copy(ones, ht.at[didx.at[j - 1]], ssem).wait()
                return 0

            lax.fori_loop(1, NCH, body, 0)
            last = NCH - 1
            pltpu.make_async_copy(ones, hs.at[sidx.at[last]], ssem).wait()
            pltpu.make_async_copy(ones, ht.at[didx.at[last]], ssem).wait()

        @pl.when(c == 0)
        def _():
            accumulate(src_ui, dst_ui)

        @pl.when(c == 1)
        def _():
            accumulate(src_iu, dst_iu)

        plsc.subcore_barrier()

        @pl.when((c == 0) & (s == 0))
        def _():
            pltpu.sync_copy(hs, d_su)

        @pl.when((c == 0) & (s == 1))
        def _():
            pltpu.sync_copy(ht, d_di)

        @pl.when((c == 1) & (s == 0))
        def _():
            pltpu.sync_copy(hs, d_si)

        @pl.when((c == 1) & (s == 1))
        def _():
            pltpu.sync_copy(ht, d_du)

    return deg_kernel


def _make_gsa_kernel(N, N1, D, NCH):
    IB = 40
    assert NCH % IB == 0
    NB = NCH // IB

    @functools.partial(
        pl.kernel,
        out_type=[
            jax.ShapeDtypeStruct((N, D), jnp.float32),
            jax.ShapeDtypeStruct((N, D), jnp.float32),
        ],
        mesh=_mesh(),
        scratch_types=[
            pltpu.VMEM((IB, _CH), jnp.int32),
            pltpu.VMEM((IB, _CH), jnp.int32),
            pltpu.VMEM((2, _CH, D), jnp.float32),
            pltpu.VMEM_SHARED((N1, D), jnp.float32),
            pltpu.SemaphoreType.DMA,
            pltpu.SemaphoreType.DMA,
        ],
    )
    def gsa_kernel(tab_ui, tab_iu, src_ui, dst_ui, src_iu, dst_iu, zblk,
                   acc_item, acc_user,
                   sidx, didx, rows, acc, gsem, ssem):
        c = lax.axis_index("c")
        s = lax.axis_index("s")

        pltpu.sync_copy(zblk, rows.at[0])
        rz = jnp.minimum(s * _ROWS, N1 - _ROWS)
        nzc = (_ROWS + _CH - 1) // _CH
        for t in range(nzc):
            off = pl.multiple_of(
                rz + jnp.minimum(t * _CH, _ROWS - _CH), 8)
            pltpu.sync_copy(rows.at[0], acc.at[pl.ds(off, _CH)])
        plsc.subcore_barrier()

        def relation(tab, src_hbm, dst_hbm):
            def outer(o, _):
                ob = pl.multiple_of(o * IB, 8)
                pltpu.sync_copy(src_hbm.at[s, pl.ds(ob, IB)], sidx)
                pltpu.sync_copy(dst_hbm.at[s, pl.ds(ob, IB)], didx)
                H = _CH // 2

                def issue_g(j, b):
                    pltpu.async_copy(tab.at[sidx.at[j, pl.ds(0, H)]],
                                     rows.at[b, pl.ds(0, H)], gsem)
                    pltpu.async_copy(tab.at[sidx.at[j, pl.ds(H, H)]],
                                     rows.at[b, pl.ds(H, H)], gsem)

                def wait_g(b):
                    for _ in range(2):
                        pltpu.make_async_copy(tab.at[sidx.at[0, pl.ds(0, H)]],
                                              rows.at[b, pl.ds(0, H)],
                                              gsem).wait()

                def issue_s(j, b):
                    pltpu.async_copy(rows.at[b], acc.at[didx.at[j]], ssem,
                                     add=True)

                def wait_s(b):
                    pltpu.make_async_copy(rows.at[b], acc.at[didx.at[0]],
                                          ssem).wait()

                issue_g(0, 0)
                wait_g(0)
                issue_s(0, 0)
                issue_g(1, 1)

                def body(h, _):
                    for bi in range(2):
                        j = 1 + h * 2 + bi
                        b = (1 + bi) % 2
                        wait_g(b)
                        issue_s(j, b)
                        wait_s(1 - b)
                        issue_g(j + 1, 1 - b)
                    return 0

                lax.fori_loop(0, (IB - 2) // 2, body, 0)
                wait_g(1)
                issue_s(IB - 1, 1)
                wait_s(0)
                wait_s(1)
                return 0

            lax.fori_loop(0, NB, outer, 0)

        @pl.when(c == 0)
        def _():
            relation(tab_ui, src_ui, dst_ui)

        @pl.when(c == 1)
        def _():
            relation(tab_iu, src_iu, dst_iu)

        plsc.subcore_barrier()

        ro = pl.multiple_of(jnp.minimum(s * _ROWS, N - _ROWS), 8)

        @pl.when(c == 0)
        def _():
            pltpu.sync_copy(acc.at[pl.ds(ro, _ROWS)],
                            acc_item.at[pl.ds(ro, _ROWS)])

        @pl.when(c == 1)
        def _():
            pltpu.sync_copy(acc.at[pl.ds(ro, _ROWS)],
                            acc_user.at[pl.ds(ro, _ROWS)])

    return gsa_kernel


def _mm_scale(x, W, deg, out_rows):
    N, D_in = x.shape
    D_out = W.shape[1]
    B = 2000
    assert N % B == 0

    def body(x_ref, w_ref, deg_ref, o_ref):
        dg = deg_ref[...]
        scale = jnp.where(dg > 0.0, lax.rsqrt(dg), 0.0)
        o_ref[...] = jnp.dot(x_ref[...], w_ref[...],
                             preferred_element_type=jnp.float32) * scale

    return pl.pallas_call(
        body,
        grid=(N // B,),
        in_specs=[
            pl.BlockSpec((B, D_in), lambda i: (i, 0)),
            pl.BlockSpec((D_in, D_out), lambda i: (0, 0)),
            pl.BlockSpec((B, 1), lambda i: (i, 0)),
        ],
        out_specs=pl.BlockSpec((B, D_out), lambda i: (i, 0)),
        out_shape=jax.ShapeDtypeStruct((out_rows, D_out), jnp.float32),
    )(x, W, deg[:, None])


def _scale_relu(acc, deg):
    N, D = acc.shape
    B = 2000
    assert N % B == 0

    def body(a_ref, deg_ref, o_ref):
        dg = deg_ref[...]
        scale = jnp.where(dg > 0.0, lax.rsqrt(dg), 0.0)
        o_ref[...] = jnp.maximum(a_ref[...] * scale, 0.0)

    return pl.pallas_call(
        body,
        grid=(N // B,),
        in_specs=[
            pl.BlockSpec((B, D), lambda i: (i, 0)),
            pl.BlockSpec((B, 1), lambda i: (i, 0)),
        ],
        out_specs=pl.BlockSpec((B, D), lambda i: (i, 0)),
        out_shape=jax.ShapeDtypeStruct((N, D), jnp.float32),
    )(acc, deg[:, None])


def kernel(x_user, x_item, edge_index_user_item, edge_index_item_user,
           W_ui_src, W_ui_tgt, W_iu_src, W_iu_tgt):
    n_user, D = x_user.shape
    n_item = x_item.shape[0]
    assert n_user == n_item
    N = n_user
    N1 = N + _PADROWS
    E = edge_index_user_item.shape[1]
    grain = _NS * _CH * 40
    Epad = ((E + grain - 1) // grain) * grain
    NCH = Epad // (_NS * _CH)

    pad = Epad - E
    sent = (jnp.arange(pad, dtype=jnp.int32) % _PADROWS) + N

    def prep(e):
        e = e.astype(jnp.int32)
        src = jnp.concatenate([e[0], sent]).reshape(_NS, NCH, _CH)
        dst = jnp.concatenate([e[1], sent]).reshape(_NS, NCH, _CH)
        return src, dst

    src_ui, dst_ui = prep(edge_index_user_item)
    src_iu, dst_iu = prep(edge_index_item_user)

    zvec = jnp.zeros((N1,), jnp.float32)
    zblk = jnp.zeros((_CH, D), jnp.float32)

    deg_kernel = _make_deg_kernel(N, N1, NCH)
    d_su, d_di, d_si, d_du = deg_kernel(src_ui, dst_ui, src_iu, dst_iu, zvec)
    d_su, d_di, d_si, d_du = (d[:N] for d in (d_su, d_di, d_si, d_du))

    tab_ui = _mm_scale(x_user, W_ui_src, d_su, N1)
    tab_iu = _mm_scale(x_item, W_iu_src, d_si, N1)

    gsa_kernel = _make_gsa_kernel(N, N1, D, NCH)
    acc_item, acc_user = gsa_kernel(tab_ui, tab_iu,
                                    src_ui, dst_ui, src_iu, dst_iu, zblk)

    out_item = _scale_relu(acc_item, d_di)
    out_user = _scale_relu(acc_user, d_du)
    return (out_user, out_item)
